# baseline (speedup 1.0000x reference)
import functools

import jax
import jax.numpy as jnp
import numpy as np
from jax.experimental import pallas as pl
from jax.experimental.pallas import tpu as pltpu

N_DIMS = 64
N_SLOTS = 512
N_READS = 4
TAU = 1.0
ALPHA = 0.7
EPS = 1e-8


_NOISE_CACHE = {}


def _rotl32(x, r):
    return (x << np.uint32(r)) | (x >> np.uint32(32 - r))


def _threefry2x32_np(k1, k2, x0, x1):
    rotations = ((13, 15, 26, 6), (17, 29, 16, 24))
    ks = (np.uint32(k1), np.uint32(k2),
          np.uint32(k1) ^ np.uint32(k2) ^ np.uint32(0x1BD11BDA))
    x0 = x0 + ks[0]
    x1 = x1 + ks[1]
    for i in range(5):
        for r in rotations[i % 2]:
            x0 = x0 + x1
            x1 = _rotl32(x1, r)
            x1 = x1 ^ x0
        x0 = x0 + ks[(i + 1) % 3]
        x1 = x1 + ks[(i + 2) % 3] + np.uint32(i + 1)
    return x0, x1


def _uniform_np(seed, shape):
    n = int(np.prod(shape))
    c1 = np.zeros(n, dtype=np.uint32)
    c2 = np.arange(n, dtype=np.uint32)
    x0, x1 = _threefry2x32_np(0, seed, c1, c2)
    bits = x0 ^ x1
    floats = ((bits >> np.uint32(9)) | np.uint32(0x3F800000)).view(np.float32)
    floats = floats - np.float32(1.0)
    lo, hi = np.float32(1e-20), np.float32(1.0)
    return np.maximum(lo, floats * (hi - lo) + lo).reshape(shape)


def _gumbel_consts(bs):
    if bs not in _NOISE_CACHE:
        uw = _uniform_np(12345, (bs, 1, N_SLOTS)).astype(np.float64)
        gw = (-np.log(-np.log(uw))).astype(np.float32).reshape(bs, N_SLOTS)
        ur = _uniform_np(54321, (bs, N_READS, N_SLOTS)).astype(np.float64)
        gr = (-np.log(-np.log(ur))).astype(np.float32)
        _NOISE_CACHE[bs] = (gw, gr)
    return _NOISE_CACHE[bs]


def _softmax_last(x):
    m = jnp.max(x, axis=-1, keepdims=True)
    e = jnp.exp(x - m)
    return e / jnp.sum(e, axis=-1, keepdims=True)


def _body(kr_ref, mem_ref, usage_ref, rw_ref, mt_ref, hx_ref, midx_ref,
          wg_ref, bg_ref, seq_ref, gw_ref, gr_ref,
          mread_ref, mtout_ref, usage_out_ref, readwt_ref, merased_ref,
          midx_out_ref):
    B = mem_ref.shape[0]
    HI = jax.lax.Precision.DEFAULT
    M = mem_ref[...]
    BF = jnp.bfloat16
    Mb = M.astype(BF)
    wm = jnp.tanh(mt_ref[...])
    ones_w = jnp.ones((B, N_DIMS), BF)

    col_sq = jax.lax.dot_general(ones_w, Mb * Mb,
                                 (((1,), (1,)), ((0,), (0,))),
                                 precision=HI,
                                 preferred_element_type=jnp.float32)
    col_n = jnp.sqrt(col_sq)
    wm_n = jnp.sqrt(jnp.sum(wm * wm, axis=-1, keepdims=True))
    wmn = wm / (wm_n + EPS)
    dot = jax.lax.dot_general(wmn.astype(BF), Mb,
                              (((1,), (1,)), ((0,), (0,))),
                              precision=HI,
                              preferred_element_type=jnp.float32)
    dist = -(dot / (col_n + EPS))
    soft = _softmax_last(dist)

    gamma = jax.nn.sigmoid(
        jnp.dot(hx_ref[...], wg_ref[...],
                preferred_element_type=jnp.float32) + bg_ref[0, 0])
    u_t = ALPHA * usage_ref[...] + (1.0 - ALPHA) * jnp.sum(rw_ref[...], axis=1)
    soft = soft + gamma * u_t

    wt = _softmax_last((1.0 - soft) + gw_ref[...])

    merased_ref[...] = jax.lax.dot_general(
        wt.astype(BF), Mb, (((1,), (2,)), ((0,), (0,))), precision=HI,
        preferred_element_type=jnp.float32)[:, None, :]
    Mt = M * (1.0 - wt)[:, None, :] + wm[:, :, None] * wt[:, None, :]
    mtout_ref[...] = Mt
    usage_out_ref[...] = u_t * (1.0 - wt)
    midx_out_ref[...] = jnp.where(wt == 1.0, seq_ref[0, 0], midx_ref[...])

    rk = jnp.tanh(kr_ref[...]).reshape(B, N_READS, N_DIMS)
    rk_n = jnp.sqrt(jnp.sum(rk * rk, axis=-1, keepdims=True))
    rkn = rk / (rk_n + EPS)
    dotw = dot * (wm_n + EPS)
    wm_sq = jnp.sum(wm * wm, axis=-1, keepdims=True)
    omw = 1.0 - wt
    mt_sq = omw * omw * col_sq + 2.0 * wt * omw * dotw + wt * wt * wm_sq
    inv_mt_n = 1.0 / (jnp.sqrt(mt_sq) + EPS)

    Mtb = Mt.astype(BF)
    dotr = jax.lax.dot_general(rkn.astype(BF), Mtb,
                               (((2,), (1,)), ((0,), (0,))),
                               precision=HI,
                               preferred_element_type=jnp.float32)
    dist_r = dotr * inv_mt_n[:, None, :]
    read_wt = _softmax_last(dist_r + gr_ref[...])
    readwt_ref[...] = read_wt
    mread_ref[...] = jax.lax.dot_general(
        read_wt.astype(BF), Mtb, (((2,), (2,)), ((0,), (0,))),
        precision=HI, preferred_element_type=jnp.float32)


@functools.partial(jax.jit, static_argnames=())
def kernel(k_r, memory, usage, read_weights, m_t, hx, m_idx, seq_idx,
           W_gate, b_gate):
    bs = k_r.shape[0]
    gw, gr = _gumbel_consts(bs)

    hx2 = hx[-1]
    seqf = jnp.asarray(seq_idx, jnp.float32).reshape(1, 1)
    bg = jnp.asarray(b_gate, jnp.float32).reshape(1, 1)

    B = 32
    grid = (bs // B,)
    hidden = hx2.shape[1]

    out = pl.pallas_call(
        _body,
        grid=grid,
        in_specs=[
            pl.BlockSpec((B, N_READS * N_DIMS), lambda i: (i, 0)),
            pl.BlockSpec((B, N_DIMS, N_SLOTS), lambda i: (i, 0, 0)),
            pl.BlockSpec((B, N_SLOTS), lambda i: (i, 0)),
            pl.BlockSpec((B, N_READS, N_SLOTS), lambda i: (i, 0, 0)),
            pl.BlockSpec((B, N_DIMS), lambda i: (i, 0)),
            pl.BlockSpec((B, hidden), lambda i: (i, 0)),
            pl.BlockSpec((B, N_SLOTS), lambda i: (i, 0)),
            pl.BlockSpec((hidden, 1), lambda i: (0, 0)),
            pl.BlockSpec((1, 1), lambda i: (0, 0)),
            pl.BlockSpec((1, 1), lambda i: (0, 0)),
            pl.BlockSpec((B, N_SLOTS), lambda i: (i, 0)),
            pl.BlockSpec((B, N_READS, N_SLOTS), lambda i: (i, 0, 0)),
        ],
        out_specs=[
            pl.BlockSpec((B, N_READS, N_DIMS), lambda i: (i, 0, 0)),
            pl.BlockSpec((B, N_DIMS, N_SLOTS), lambda i: (i, 0, 0)),
            pl.BlockSpec((B, N_SLOTS), lambda i: (i, 0)),
            pl.BlockSpec((B, N_READS, N_SLOTS), lambda i: (i, 0, 0)),
            pl.BlockSpec((B, 1, N_DIMS), lambda i: (i, 0, 0)),
            pl.BlockSpec((B, N_SLOTS), lambda i: (i, 0)),
        ],
        out_shape=[
            jax.ShapeDtypeStruct((bs, N_READS, N_DIMS), jnp.float32),
            jax.ShapeDtypeStruct((bs, N_DIMS, N_SLOTS), jnp.float32),
            jax.ShapeDtypeStruct((bs, N_SLOTS), jnp.float32),
            jax.ShapeDtypeStruct((bs, N_READS, N_SLOTS), jnp.float32),
            jax.ShapeDtypeStruct((bs, 1, N_DIMS), jnp.float32),
            jax.ShapeDtypeStruct((bs, N_SLOTS), jnp.float32),
        ],
        compiler_params=pltpu.CompilerParams(
            dimension_semantics=("parallel",),
        ),
    )(k_r, memory, usage, read_weights, m_t, hx2, m_idx, W_gate, bg, seqf,
      gw, gr)

    m_read, Mt, usage_new, read_wt, m_erased, m_idx_new = out
    return (m_read, Mt, usage_new, read_wt, m_erased, m_idx_new)


# speedup vs baseline: 1.0568x; 1.0568x over previous
import functools

import jax
import jax.numpy as jnp
import numpy as np
from jax.experimental import pallas as pl
from jax.experimental.pallas import tpu as pltpu

N_DIMS = 64
N_SLOTS = 512
N_READS = 4
TAU = 1.0
ALPHA = 0.7
EPS = 1e-8


_NOISE_CACHE = {}


def _rotl32(x, r):
    return (x << np.uint32(r)) | (x >> np.uint32(32 - r))


def _threefry2x32_np(k1, k2, x0, x1):
    rotations = ((13, 15, 26, 6), (17, 29, 16, 24))
    ks = (np.uint32(k1), np.uint32(k2),
          np.uint32(k1) ^ np.uint32(k2) ^ np.uint32(0x1BD11BDA))
    x0 = x0 + ks[0]
    x1 = x1 + ks[1]
    for i in range(5):
        for r in rotations[i % 2]:
            x0 = x0 + x1
            x1 = _rotl32(x1, r)
            x1 = x1 ^ x0
        x0 = x0 + ks[(i + 1) % 3]
        x1 = x1 + ks[(i + 2) % 3] + np.uint32(i + 1)
    return x0, x1


def _uniform_np(seed, shape):
    n = int(np.prod(shape))
    c1 = np.zeros(n, dtype=np.uint32)
    c2 = np.arange(n, dtype=np.uint32)
    x0, x1 = _threefry2x32_np(0, seed, c1, c2)
    bits = x0 ^ x1
    floats = ((bits >> np.uint32(9)) | np.uint32(0x3F800000)).view(np.float32)
    floats = floats - np.float32(1.0)
    lo, hi = np.float32(1e-20), np.float32(1.0)
    return np.maximum(lo, floats * (hi - lo) + lo).reshape(shape)


def _gumbel_consts(bs):
    if bs not in _NOISE_CACHE:
        uw = _uniform_np(12345, (bs, 1, N_SLOTS)).astype(np.float64)
        gw = (-np.log(-np.log(uw))).astype(np.float32).reshape(bs, N_SLOTS)
        ur = _uniform_np(54321, (bs, N_READS, N_SLOTS)).astype(np.float64)
        gr = (-np.log(-np.log(ur))).astype(np.float32)
        _NOISE_CACHE[bs] = (gw, gr)
    return _NOISE_CACHE[bs]


def _softmax_last(x):
    m = jnp.max(x, axis=-1, keepdims=True)
    e = jnp.exp(x - m)
    return e / jnp.sum(e, axis=-1, keepdims=True)


def _body(kr_ref, mem_ref, usage_ref, rw_ref, mt_ref, hx_ref, midx_ref,
          wg_ref, bg_ref, seq_ref, gw_ref, gr_ref,
          mread_ref, mtout_ref, usage_out_ref, readwt_ref, merased_ref,
          midx_out_ref):
    B = mem_ref.shape[0]
    HI = jax.lax.Precision.DEFAULT
    M = mem_ref[...]
    BF = jnp.bfloat16
    Mb = M.astype(BF)
    wm = jnp.tanh(mt_ref[...])
    ones_w = jnp.ones((B, N_DIMS), BF)

    col_sq = jax.lax.dot_general(ones_w, Mb * Mb,
                                 (((1,), (1,)), ((0,), (0,))),
                                 precision=HI,
                                 preferred_element_type=jnp.float32)
    col_n = jnp.sqrt(col_sq)
    wm_n = jnp.sqrt(jnp.sum(wm * wm, axis=-1, keepdims=True))
    wmn = wm / (wm_n + EPS)
    dot = jax.lax.dot_general(wmn.astype(BF), Mb,
                              (((1,), (1,)), ((0,), (0,))),
                              precision=HI,
                              preferred_element_type=jnp.float32)
    dist = -(dot / (col_n + EPS))
    soft = _softmax_last(dist)

    gamma = jax.nn.sigmoid(
        jnp.dot(hx_ref[...], wg_ref[...],
                preferred_element_type=jnp.float32) + bg_ref[0, 0])
    u_t = ALPHA * usage_ref[...] + (1.0 - ALPHA) * jnp.sum(rw_ref[...], axis=1)
    soft = soft + gamma * u_t

    wt = _softmax_last((1.0 - soft) + gw_ref[...])

    merased_ref[...] = jax.lax.dot_general(
        wt.astype(BF), Mb, (((1,), (2,)), ((0,), (0,))), precision=HI,
        preferred_element_type=jnp.float32)[:, None, :]
    Mt = M * (1.0 - wt)[:, None, :] + wm[:, :, None] * wt[:, None, :]
    mtout_ref[...] = Mt
    usage_out_ref[...] = u_t * (1.0 - wt)
    midx_out_ref[...] = jnp.where(wt == 1.0, seq_ref[0, 0], midx_ref[...])

    rk = jnp.tanh(kr_ref[...]).reshape(B, N_READS, N_DIMS)
    rk_n = jnp.sqrt(jnp.sum(rk * rk, axis=-1, keepdims=True))
    rkn = rk / (rk_n + EPS)
    dotw = dot * (wm_n + EPS)
    wm_sq = jnp.sum(wm * wm, axis=-1, keepdims=True)
    omw = 1.0 - wt
    mt_sq = omw * omw * col_sq + 2.0 * wt * omw * dotw + wt * wt * wm_sq
    inv_mt_n = 1.0 / (jnp.sqrt(mt_sq) + EPS)

    dotr = jax.lax.dot_general(rkn, Mt, (((2,), (1,)), ((0,), (0,))),
                               precision=HI)
    dist_r = dotr * inv_mt_n[:, None, :]
    read_wt = _softmax_last(dist_r + gr_ref[...])
    readwt_ref[...] = read_wt
    mread_ref[...] = jax.lax.dot_general(
        read_wt, Mt, (((2,), (2,)), ((0,), (0,))), precision=HI)


@functools.partial(jax.jit, static_argnames=())
def kernel(k_r, memory, usage, read_weights, m_t, hx, m_idx, seq_idx,
           W_gate, b_gate):
    bs = k_r.shape[0]
    gw, gr = _gumbel_consts(bs)

    hx2 = hx[-1]
    seqf = jnp.asarray(seq_idx, jnp.float32).reshape(1, 1)
    bg = jnp.asarray(b_gate, jnp.float32).reshape(1, 1)

    B = 32
    grid = (bs // B,)
    hidden = hx2.shape[1]

    out = pl.pallas_call(
        _body,
        grid=grid,
        in_specs=[
            pl.BlockSpec((B, N_READS * N_DIMS), lambda i: (i, 0)),
            pl.BlockSpec((B, N_DIMS, N_SLOTS), lambda i: (i, 0, 0)),
            pl.BlockSpec((B, N_SLOTS), lambda i: (i, 0)),
            pl.BlockSpec((B, N_READS, N_SLOTS), lambda i: (i, 0, 0)),
            pl.BlockSpec((B, N_DIMS), lambda i: (i, 0)),
            pl.BlockSpec((B, hidden), lambda i: (i, 0)),
            pl.BlockSpec((B, N_SLOTS), lambda i: (i, 0)),
            pl.BlockSpec((hidden, 1), lambda i: (0, 0)),
            pl.BlockSpec((1, 1), lambda i: (0, 0)),
            pl.BlockSpec((1, 1), lambda i: (0, 0)),
            pl.BlockSpec((B, N_SLOTS), lambda i: (i, 0)),
            pl.BlockSpec((B, N_READS, N_SLOTS), lambda i: (i, 0, 0)),
        ],
        out_specs=[
            pl.BlockSpec((B, N_READS, N_DIMS), lambda i: (i, 0, 0)),
            pl.BlockSpec((B, N_DIMS, N_SLOTS), lambda i: (i, 0, 0)),
            pl.BlockSpec((B, N_SLOTS), lambda i: (i, 0)),
            pl.BlockSpec((B, N_READS, N_SLOTS), lambda i: (i, 0, 0)),
            pl.BlockSpec((B, 1, N_DIMS), lambda i: (i, 0, 0)),
            pl.BlockSpec((B, N_SLOTS), lambda i: (i, 0)),
        ],
        out_shape=[
            jax.ShapeDtypeStruct((bs, N_READS, N_DIMS), jnp.float32),
            jax.ShapeDtypeStruct((bs, N_DIMS, N_SLOTS), jnp.float32),
            jax.ShapeDtypeStruct((bs, N_SLOTS), jnp.float32),
            jax.ShapeDtypeStruct((bs, N_READS, N_SLOTS), jnp.float32),
            jax.ShapeDtypeStruct((bs, 1, N_DIMS), jnp.float32),
            jax.ShapeDtypeStruct((bs, N_SLOTS), jnp.float32),
        ],
        compiler_params=pltpu.CompilerParams(
            dimension_semantics=("parallel",),
        ),
    )(k_r, memory, usage, read_weights, m_t, hx2, m_idx, W_gate, bg, seqf,
      gw, gr)

    m_read, Mt, usage_new, read_wt, m_erased, m_idx_new = out
    return (m_read, Mt, usage_new, read_wt, m_erased, m_idx_new)


# speedup vs baseline: 1.0680x; 1.0106x over previous
import functools

import jax
import jax.numpy as jnp
import numpy as np
from jax.experimental import pallas as pl
from jax.experimental.pallas import tpu as pltpu

N_DIMS = 64
N_SLOTS = 512
N_READS = 4
TAU = 1.0
ALPHA = 0.7
EPS = 1e-8


_NOISE_CACHE = {}


def _rotl32(x, r):
    return (x << np.uint32(r)) | (x >> np.uint32(32 - r))


def _threefry2x32_np(k1, k2, x0, x1):
    rotations = ((13, 15, 26, 6), (17, 29, 16, 24))
    ks = (np.uint32(k1), np.uint32(k2),
          np.uint32(k1) ^ np.uint32(k2) ^ np.uint32(0x1BD11BDA))
    x0 = x0 + ks[0]
    x1 = x1 + ks[1]
    for i in range(5):
        for r in rotations[i % 2]:
            x0 = x0 + x1
            x1 = _rotl32(x1, r)
            x1 = x1 ^ x0
        x0 = x0 + ks[(i + 1) % 3]
        x1 = x1 + ks[(i + 2) % 3] + np.uint32(i + 1)
    return x0, x1


def _uniform_np(seed, shape):
    n = int(np.prod(shape))
    c1 = np.zeros(n, dtype=np.uint32)
    c2 = np.arange(n, dtype=np.uint32)
    x0, x1 = _threefry2x32_np(0, seed, c1, c2)
    bits = x0 ^ x1
    floats = ((bits >> np.uint32(9)) | np.uint32(0x3F800000)).view(np.float32)
    floats = floats - np.float32(1.0)
    lo, hi = np.float32(1e-20), np.float32(1.0)
    return np.maximum(lo, floats * (hi - lo) + lo).reshape(shape)


def _gumbel_consts(bs):
    if bs not in _NOISE_CACHE:
        uw = _uniform_np(12345, (bs, 1, N_SLOTS)).astype(np.float64)
        gw = (-np.log(-np.log(uw))).astype(np.float32).reshape(bs, N_SLOTS)
        ur = _uniform_np(54321, (bs, N_READS, N_SLOTS)).astype(np.float64)
        gr = (-np.log(-np.log(ur))).astype(np.float32)
        _NOISE_CACHE[bs] = (gw, gr)
    return _NOISE_CACHE[bs]


def _softmax_last(x):
    m = jnp.max(x, axis=-1, keepdims=True)
    e = jnp.exp(x - m)
    return e / jnp.sum(e, axis=-1, keepdims=True)


def _body(kr_ref, mem_ref, usage_ref, rw_ref, mt_ref, hx_ref, midx_ref,
          wg_ref, bg_ref, seq_ref, gw_ref, gr_ref,
          mread_ref, mtout_ref, usage_out_ref, readwt_ref, merased_ref,
          midx_out_ref):
    B = mem_ref.shape[0]
    HI = jax.lax.Precision.DEFAULT
    M = mem_ref[...]
    BF = jnp.bfloat16
    Mb = M.astype(BF)
    wm = jnp.tanh(mt_ref[...])
    ones_w = jnp.ones((B, N_DIMS), BF)

    col_sq = jax.lax.dot_general(ones_w, Mb * Mb,
                                 (((1,), (1,)), ((0,), (0,))),
                                 precision=HI,
                                 preferred_element_type=jnp.float32)
    col_n = jnp.sqrt(col_sq)
    wm_n = jnp.sqrt(jnp.sum(wm * wm, axis=-1, keepdims=True))
    wmn = wm / (wm_n + EPS)
    dot = jax.lax.dot_general(wmn.astype(BF), Mb,
                              (((1,), (1,)), ((0,), (0,))),
                              precision=HI,
                              preferred_element_type=jnp.float32)
    dist = -(dot / (col_n + EPS))
    soft = _softmax_last(dist)

    gamma = jax.nn.sigmoid(
        jnp.dot(hx_ref[...], wg_ref[...],
                preferred_element_type=jnp.float32) + bg_ref[0, 0])
    r_w = jax.lax.dot_general(jnp.ones((B, N_READS), BF),
                              rw_ref[...].astype(BF),
                              (((1,), (1,)), ((0,), (0,))),
                              precision=HI,
                              preferred_element_type=jnp.float32)
    u_t = ALPHA * usage_ref[...] + (1.0 - ALPHA) * r_w
    soft = soft + gamma * u_t

    wt = _softmax_last((1.0 - soft) + gw_ref[...])

    merased_ref[...] = jax.lax.dot_general(
        wt.astype(BF), Mb, (((1,), (2,)), ((0,), (0,))), precision=HI,
        preferred_element_type=jnp.float32)[:, None, :]
    Mt = M * (1.0 - wt)[:, None, :] + wm[:, :, None] * wt[:, None, :]
    mtout_ref[...] = Mt
    usage_out_ref[...] = u_t * (1.0 - wt)
    midx_out_ref[...] = jnp.where(wt == 1.0, seq_ref[0, 0], midx_ref[...])

    rk = jnp.tanh(kr_ref[...]).reshape(B, N_READS, N_DIMS)
    rk_n = jnp.sqrt(jnp.sum(rk * rk, axis=-1, keepdims=True))
    rkn = rk / (rk_n + EPS)
    dotw = dot * (wm_n + EPS)
    wm_sq = jnp.sum(wm * wm, axis=-1, keepdims=True)
    omw = 1.0 - wt
    mt_sq = omw * omw * col_sq + 2.0 * wt * omw * dotw + wt * wt * wm_sq
    inv_mt_n = 1.0 / (jnp.sqrt(mt_sq) + EPS)

    dotr = jax.lax.dot_general(rkn, Mt, (((2,), (1,)), ((0,), (0,))),
                               precision=HI)
    dist_r = dotr * inv_mt_n[:, None, :]
    read_wt = _softmax_last(dist_r + gr_ref[...])
    readwt_ref[...] = read_wt
    mread_ref[...] = jax.lax.dot_general(
        read_wt, Mt, (((2,), (2,)), ((0,), (0,))), precision=HI)


@functools.partial(jax.jit, static_argnames=())
def kernel(k_r, memory, usage, read_weights, m_t, hx, m_idx, seq_idx,
           W_gate, b_gate):
    bs = k_r.shape[0]
    gw, gr = _gumbel_consts(bs)

    hx2 = hx[-1]
    seqf = jnp.asarray(seq_idx, jnp.float32).reshape(1, 1)
    bg = jnp.asarray(b_gate, jnp.float32).reshape(1, 1)

    B = 32
    grid = (bs // B,)
    hidden = hx2.shape[1]

    out = pl.pallas_call(
        _body,
        grid=grid,
        in_specs=[
            pl.BlockSpec((B, N_READS * N_DIMS), lambda i: (i, 0)),
            pl.BlockSpec((B, N_DIMS, N_SLOTS), lambda i: (i, 0, 0)),
            pl.BlockSpec((B, N_SLOTS), lambda i: (i, 0)),
            pl.BlockSpec((B, N_READS, N_SLOTS), lambda i: (i, 0, 0)),
            pl.BlockSpec((B, N_DIMS), lambda i: (i, 0)),
            pl.BlockSpec((B, hidden), lambda i: (i, 0)),
            pl.BlockSpec((B, N_SLOTS), lambda i: (i, 0)),
            pl.BlockSpec((hidden, 1), lambda i: (0, 0)),
            pl.BlockSpec((1, 1), lambda i: (0, 0)),
            pl.BlockSpec((1, 1), lambda i: (0, 0)),
            pl.BlockSpec((B, N_SLOTS), lambda i: (i, 0)),
            pl.BlockSpec((B, N_READS, N_SLOTS), lambda i: (i, 0, 0)),
        ],
        out_specs=[
            pl.BlockSpec((B, N_READS, N_DIMS), lambda i: (i, 0, 0)),
            pl.BlockSpec((B, N_DIMS, N_SLOTS), lambda i: (i, 0, 0)),
            pl.BlockSpec((B, N_SLOTS), lambda i: (i, 0)),
            pl.BlockSpec((B, N_READS, N_SLOTS), lambda i: (i, 0, 0)),
            pl.BlockSpec((B, 1, N_DIMS), lambda i: (i, 0, 0)),
            pl.BlockSpec((B, N_SLOTS), lambda i: (i, 0)),
        ],
        out_shape=[
            jax.ShapeDtypeStruct((bs, N_READS, N_DIMS), jnp.float32),
            jax.ShapeDtypeStruct((bs, N_DIMS, N_SLOTS), jnp.float32),
            jax.ShapeDtypeStruct((bs, N_SLOTS), jnp.float32),
            jax.ShapeDtypeStruct((bs, N_READS, N_SLOTS), jnp.float32),
            jax.ShapeDtypeStruct((bs, 1, N_DIMS), jnp.float32),
            jax.ShapeDtypeStruct((bs, N_SLOTS), jnp.float32),
        ],
        compiler_params=pltpu.CompilerParams(
            dimension_semantics=("parallel",),
        ),
    )(k_r, memory, usage, read_weights, m_t, hx2, m_idx, W_gate, bg, seqf,
      gw, gr)

    m_read, Mt, usage_new, read_wt, m_erased, m_idx_new = out
    return (m_read, Mt, usage_new, read_wt, m_erased, m_idx_new)
